# Initial kernel scaffold; baseline (speedup 1.0000x reference)
import jax
import jax.numpy as jnp
from jax.experimental import pallas as pl


def kernel(token_ids, weights):
    raise NotImplementedError("write your pallas kernel here")


# speedup vs baseline: 1.4589x; 1.4589x over previous
import functools

import jax
import jax.numpy as jnp
from jax import lax
from jax.experimental import pallas as pl
from jax.experimental.pallas import tpu as pltpu
from jax.experimental.pallas import tpu_sc as plsc

_NUM_CORES = 2
_NUM_SUBCORES = 16
_NUM_WORKERS = _NUM_CORES * _NUM_SUBCORES

_CHUNK = 1024


def _make_lookup(total: int, vocab: int, dim: int):
    assert total % (_NUM_WORKERS * _CHUNK) == 0
    per_worker = total // _NUM_WORKERS
    num_chunks = per_worker // _CHUNK

    mesh = plsc.VectorSubcoreMesh(core_axis_name="c", subcore_axis_name="s")

    @functools.partial(
        pl.kernel,
        mesh=mesh,
        out_type=jax.ShapeDtypeStruct((total, dim), jnp.float32),
        scratch_types=[
            pltpu.VMEM((_CHUNK,), jnp.int32),
            pltpu.VMEM((_CHUNK, dim), jnp.float32),
            pltpu.SemaphoreType.DMA,
        ],
        compiler_params=pltpu.CompilerParams(use_tc_tiling_on_sc=False),
    )
    def lookup(idx_hbm, table_hbm, out_hbm, idx_v, rows_v, sem):
        wid = lax.axis_index("s") * _NUM_CORES + lax.axis_index("c")
        base = wid * per_worker

        def chunk_body(j, carry):
            off = base + j * _CHUNK
            pltpu.sync_copy(idx_hbm.at[pl.ds(off, _CHUNK)], idx_v)
            pltpu.async_copy(table_hbm.at[idx_v], rows_v, sem).wait()
            pltpu.sync_copy(rows_v, out_hbm.at[pl.ds(off, _CHUNK)])
            return carry

        lax.fori_loop(0, num_chunks, chunk_body, 0)

    return lookup


def kernel(token_ids, weights):
    batch, seq = token_ids.shape
    vocab, dim = weights.shape
    total = batch * seq
    flat_idx = token_ids.reshape(total).astype(jnp.int32)
    lookup = _make_lookup(total, vocab, dim)
    out = lookup(flat_idx, weights)
    return out.reshape(batch, seq, dim)


# speedup vs baseline: 1.4930x; 1.0234x over previous
import functools

import jax
import jax.numpy as jnp
from jax import lax
from jax.experimental import pallas as pl
from jax.experimental.pallas import tpu as pltpu
from jax.experimental.pallas import tpu_sc as plsc

_NUM_CORES = 2
_NUM_SUBCORES = 16
_NUM_WORKERS = _NUM_CORES * _NUM_SUBCORES

_CHUNK = 640
_NBUF = 4


def _make_lookup(total: int, vocab: int, dim: int):
    per_worker = total // _NUM_WORKERS
    assert per_worker % (_CHUNK * _NBUF) == 0
    num_chunks = per_worker // _CHUNK
    steps = num_chunks // _NBUF

    mesh = plsc.VectorSubcoreMesh(core_axis_name="c", subcore_axis_name="s")

    @functools.partial(
        pl.kernel,
        mesh=mesh,
        out_type=jax.ShapeDtypeStruct((total, dim), jnp.float32),
        scratch_types=[
            pltpu.VMEM((per_worker,), jnp.int32),
            [pltpu.VMEM((_CHUNK, dim), jnp.float32) for _ in range(_NBUF)],
            [pltpu.SemaphoreType.DMA for _ in range(_NBUF)],
            [pltpu.SemaphoreType.DMA for _ in range(_NBUF)],
        ],
        compiler_params=pltpu.CompilerParams(use_tc_tiling_on_sc=False),
    )
    def lookup(idx_hbm, table_hbm, out_hbm, idx_v, rows, gsems, osems):
        wid = lax.axis_index("s") * _NUM_CORES + lax.axis_index("c")
        base = wid * per_worker
        pltpu.sync_copy(idx_hbm.at[pl.ds(base, per_worker)], idx_v)

        def start_gather(j, b):
            pltpu.async_copy(
                table_hbm.at[idx_v.at[pl.ds(j * _CHUNK, _CHUNK)]],
                rows[b], gsems[b])

        def wait_gather(j, b):
            pltpu.make_async_copy(
                table_hbm.at[idx_v.at[pl.ds(j * _CHUNK, _CHUNK)]],
                rows[b], gsems[b]).wait()

        def start_out(j, b):
            pltpu.async_copy(
                rows[b], out_hbm.at[pl.ds(base + j * _CHUNK, _CHUNK)],
                osems[b])

        def wait_out(j, b):
            pltpu.make_async_copy(
                rows[b], out_hbm.at[pl.ds(base + j * _CHUNK, _CHUNK)],
                osems[b]).wait()

        for b in range(_NBUF):
            start_gather(b, b)

        def outer(step, carry):
            for b in range(_NBUF):
                j = step * _NBUF + b
                wait_gather(j, b)
                start_out(j, b)
            for b in range(_NBUF):
                j = step * _NBUF + b
                wait_out(j, b)
                start_gather(j + _NBUF, b)
            return carry

        lax.fori_loop(0, steps - 1, outer, 0)

        for b in range(_NBUF):
            j = (steps - 1) * _NBUF + b
            wait_gather(j, b)
            start_out(j, b)
        for b in range(_NBUF):
            j = (steps - 1) * _NBUF + b
            wait_out(j, b)

    return lookup


def kernel(token_ids, weights):
    batch, seq = token_ids.shape
    vocab, dim = weights.shape
    total = batch * seq
    flat_idx = token_ids.reshape(total).astype(jnp.int32)
    lookup = _make_lookup(total, vocab, dim)
    out = lookup(flat_idx, weights)
    return out.reshape(batch, seq, dim)
